# baseline (speedup 1.0000x reference)
import functools

import jax
import jax.numpy as jnp
from jax import lax
from jax.experimental import pallas as pl
from jax.experimental.pallas import tpu as pltpu
from jax.experimental.pallas import tpu_sc as plsc

VOCAB = 42000
D = 512
T = 50
TF = 48
TP = 64
NB = 4096
NC = 2
NS = 16
NW = NC * NS
BPW = NB // NW
NPAIR = BPW // 2

_mesh = plsc.VectorSubcoreMesh(core_axis_name="c", subcore_axis_name="s")


@functools.partial(
    pl.kernel,
    out_type=jax.ShapeDtypeStruct((NB, T, D), jnp.float32),
    mesh=_mesh,
    scratch_types=[
        pltpu.VMEM((BPW * TP,), jnp.int32),
        pltpu.VMEM((2, T, D), jnp.float32),
        pltpu.VMEM((2, 8, D), jnp.float32),
        pltpu.SemaphoreType.DMA((2,)),
        pltpu.SemaphoreType.DMA((2,)),
        pltpu.SemaphoreType.DMA((2,)),
    ],
)
def _gather_kernel(idx_hbm, table_hbm, out_hbm, idx_v, rows_v, tail_v,
                   gsem, tsem, osem):
    wid = lax.axis_index("s") * NC + lax.axis_index("c")
    base = wid * BPW
    pltpu.sync_copy(idx_hbm.at[pl.ds(base * TP, BPW * TP)], idx_v)

    def start_gather(j, b):
        main = pltpu.async_copy(
            table_hbm.at[idx_v.at[pl.ds(j * TP, TF)]],
            rows_v.at[b, pl.ds(0, TF)],
            gsem.at[b],
        )
        tail = pltpu.async_copy(
            table_hbm.at[idx_v.at[pl.ds(j * TP + TF, 8)]],
            tail_v.at[b],
            tsem.at[b],
        )
        return main, tail

    def fixup(b):
        for r in range(T - TF):
            for c in range(D // 16):
                rows_v[b, TF + r, pl.ds(c * 16, 16)] = (
                    tail_v[b, r, pl.ds(c * 16, 16)]
                )

    def start_out(j, b):
        pltpu.async_copy(rows_v.at[b], out_hbm.at[base + j], osem.at[b])

    def wait_out(b):
        pltpu.make_async_copy(rows_v.at[b], out_hbm.at[base], osem.at[b]).wait()

    def run_step(j, b, handles):
        ma, ta = handles
        ma.wait()
        ta.wait()
        fixup(b)
        start_out(j, b)

    h0 = start_gather(0, 0)
    h1 = start_gather(1, 1)
    run_step(0, 0, h0)
    run_step(1, 1, h1)

    def body(g, carry):
        j = 2 * g
        wait_out(0)
        ha = start_gather(j, 0)
        wait_out(1)
        hb = start_gather(j + 1, 1)
        run_step(j, 0, ha)
        run_step(j + 1, 1, hb)
        return carry

    lax.fori_loop(1, NPAIR, body, 0)

    wait_out(0)
    wait_out(1)


def kernel(x, table):
    idx = jnp.pad(x.astype(jnp.int32), ((0, 0), (0, TP - T))).reshape(NB * TP)
    return _gather_kernel(idx, table)


# speedup vs baseline: 3.0934x; 3.0934x over previous
import functools

import jax
import jax.numpy as jnp
from jax import lax
from jax.experimental import pallas as pl
from jax.experimental.pallas import tpu as pltpu
from jax.experimental.pallas import tpu_sc as plsc

VOCAB = 42000
D = 512
T = 50
TF = 48
TP = 64
NB = 4096
NC = 2
NS = 16
NW = NC * NS
BPW = NB // NW
NBUF = 3
NGRP = (BPW + NBUF - 1) // NBUF

_mesh = plsc.VectorSubcoreMesh(core_axis_name="c", subcore_axis_name="s")


@functools.partial(
    pl.kernel,
    out_type=jax.ShapeDtypeStruct((NB, T, D), jnp.float32),
    mesh=_mesh,
    scratch_types=[
        pltpu.VMEM((BPW * TP,), jnp.int32),
        pltpu.VMEM((NBUF, T, D), jnp.float32),
        pltpu.VMEM((NBUF, 8, D), jnp.float32),
        pltpu.SemaphoreType.DMA((NBUF,)),
        pltpu.SemaphoreType.DMA((NBUF,)),
        pltpu.SemaphoreType.DMA((NBUF,)),
    ],
)
def _gather_kernel(idx_hbm, table_hbm, out_hbm, idx_v, rows_v, tail_v,
                   gsem, tsem, osem):
    wid = lax.axis_index("s") * NC + lax.axis_index("c")
    base = wid * BPW
    pltpu.sync_copy(idx_hbm.at[pl.ds(base * TP, BPW * TP)], idx_v)

    def start_gather(j, b):
        main = pltpu.async_copy(
            table_hbm.at[idx_v.at[pl.ds(j * TP, TF)]],
            rows_v.at[b, pl.ds(0, TF)],
            gsem.at[b],
        )
        tail = pltpu.async_copy(
            table_hbm.at[idx_v.at[pl.ds(j * TP + TF, 8)]],
            tail_v.at[b],
            tsem.at[b],
        )
        return main, tail

    def fixup(b):
        for r in range(T - TF):
            for c in range(D // 16):
                rows_v[b, TF + r, pl.ds(c * 16, 16)] = (
                    tail_v[b, r, pl.ds(c * 16, 16)]
                )

    def start_out(j, b):
        pltpu.async_copy(rows_v.at[b], out_hbm.at[base + j], osem.at[b])

    def wait_out(b):
        pltpu.make_async_copy(rows_v.at[b], out_hbm.at[base], osem.at[b]).wait()

    def group(j0, first):
        handles = []
        for b in range(NBUF):
            if not first:
                wait_out(b)
            handles.append(start_gather(jnp.minimum(j0 + b, BPW - 1), b))
        for b in range(NBUF):
            ma, ta = handles[b]
            ma.wait()
            ta.wait()
            fixup(b)
            start_out(jnp.minimum(j0 + b, BPW - 1), b)

    group(0, True)

    def body(g, carry):
        group(g * NBUF, False)
        return carry

    lax.fori_loop(1, NGRP, body, 0)

    for b in range(NBUF):
        wait_out(b)


def kernel(x, table):
    idxp = jnp.concatenate([x.astype(jnp.int32),
                            x[:, T - (TP - T):].astype(jnp.int32)], axis=1)
    return _gather_kernel(idxp.reshape(NB * TP), table)
